# Initial kernel scaffold; baseline (speedup 1.0000x reference)
import jax
import jax.numpy as jnp
from jax.experimental import pallas as pl


def kernel(x, pairlist, f_ij, f_ij_cutoff, W_in, Wf1, bf1, Wf2, bf2, Wo1, bo1, Wo2, bo2):
    raise NotImplementedError("write your pallas kernel here")


# speedup vs baseline: 2.5521x; 2.5521x over previous
import functools

import jax
import jax.numpy as jnp
from jax import lax
from jax.experimental import pallas as pl
from jax.experimental.pallas import tpu as pltpu
from jax.experimental.pallas import tpu_sc as plsc

_N = 10000
_E = 320000
_D = 128
_F = 128
_R = 16

_NC = 2
_NS = 16
_NW = _NC * _NS
_EPW = _E // _NW
_CHUNK = 80
_NIT = _EPW // _CHUNK
_NP = 10240
_RPT = _NP // _NS

_LOG2 = 0.6931471805599453


def _ssp(v):
    return jax.nn.softplus(v) - _LOG2


def _h_body(x_ref, w_ref, o_ref):
    o_ref[...] = lax.dot_general(
        x_ref[...], w_ref[...], (((1,), (1,)), ((), ())),
        preferred_element_type=jnp.float32)


def _compute_h(x, W_in):
    return pl.pallas_call(
        _h_body,
        out_shape=jax.ShapeDtypeStruct((_N, _D), jnp.float32),
    )(x, W_in)


_BE = 8000


def _wc_body(f_ref, c_ref, w1_ref, b1_ref, w2_ref, b2_ref, o_ref):
    t = lax.dot_general(f_ref[...], w1_ref[...], (((1,), (1,)), ((), ())),
                        preferred_element_type=jnp.float32)
    t = _ssp(t + b1_ref[...])
    w = lax.dot_general(t, w2_ref[...], (((1,), (1,)), ((), ())),
                        preferred_element_type=jnp.float32)
    o_ref[...] = (w + b2_ref[...]) * c_ref[...]


def _compute_wc(f2d, cut, Wf1, bf1, Wf2, bf2):
    grid = (_E // _BE,)
    return pl.pallas_call(
        _wc_body,
        grid=grid,
        in_specs=[
            pl.BlockSpec((_BE, _R), lambda i: (i, 0)),
            pl.BlockSpec((_BE, 1), lambda i: (i, 0)),
            pl.BlockSpec((_F, _R), lambda i: (0, 0)),
            pl.BlockSpec((1, _F), lambda i: (0, 0)),
            pl.BlockSpec((_F, _F), lambda i: (0, 0)),
            pl.BlockSpec((1, _F), lambda i: (0, 0)),
        ],
        out_specs=pl.BlockSpec((_BE, _F), lambda i: (i, 0)),
        out_shape=jax.ShapeDtypeStruct((_E, _F), jnp.float32),
    )(f2d, cut, Wf1, bf1, Wf2, bf2)


def _sc_aggregate(h, wc, idx_i, idx_j, zeros):
    mesh = plsc.VectorSubcoreMesh(core_axis_name="c", subcore_axis_name="s")

    @functools.partial(
        pl.kernel,
        out_type=jax.ShapeDtypeStruct((_NC * _NP, _D), jnp.float32),
        mesh=mesh,
        scratch_types=[
            pltpu.VMEM((_CHUNK,), jnp.int32),
            pltpu.VMEM((_CHUNK,), jnp.int32),
            pltpu.VMEM((_CHUNK, _D), jnp.float32),
            pltpu.VMEM((_CHUNK, _D), jnp.float32),
            pltpu.VMEM_SHARED((_NP, _D), jnp.float32),
            pltpu.SemaphoreType.DMA,
        ],
    )
    def k(h_hbm, wc_hbm, ii_hbm, ij_hbm, z_hbm, out_hbm,
          ii_v, ij_v, rows_v, w_v, acc_sh, sem):
        cid = lax.axis_index("c")
        sid = lax.axis_index("s")
        wid = sid * _NC + cid

        pltpu.sync_copy(z_hbm, acc_sh.at[pl.ds(sid * _RPT, _RPT)])
        plsc.subcore_barrier()

        @pl.loop(0, _NIT)
        def _(it):
            base = wid * _EPW + it * _CHUNK
            pltpu.sync_copy(ii_hbm.at[pl.ds(base, _CHUNK)], ii_v)
            pltpu.sync_copy(ij_hbm.at[pl.ds(base, _CHUNK)], ij_v)
            pltpu.sync_copy(wc_hbm.at[pl.ds(base, _CHUNK)], w_v)
            pltpu.async_copy(h_hbm.at[ij_v], rows_v, sem).wait()

            @pl.loop(0, _CHUNK)
            def _(e):
                for j in range(0, _D, 16):
                    slc = (pl.ds(e, 1), pl.ds(j, 16))
                    rows_v.at[slc][...] = rows_v.at[slc][...] * w_v.at[slc][...]

            pltpu.sync_copy(rows_v, acc_sh.at[ii_v], add=True)

        plsc.subcore_barrier()
        pltpu.sync_copy(acc_sh.at[pl.ds(sid * _RPT, _RPT)],
                        out_hbm.at[pl.ds(cid * _NP + sid * _RPT, _RPT)])

    return k(h, wc, idx_i, idx_j, zeros)


def _out_body(p_ref, w1_ref, b1_ref, w2_ref, b2_ref, o_ref):
    agg = p_ref[0, :_N, :] + p_ref[1, :_N, :]
    t = lax.dot_general(agg, w1_ref[...], (((1,), (1,)), ((), ())),
                        preferred_element_type=jnp.float32)
    t = _ssp(t + b1_ref[...])
    o = lax.dot_general(t, w2_ref[...], (((1,), (1,)), ((), ())),
                        preferred_element_type=jnp.float32)
    o_ref[...] = o + b2_ref[...]


def _out_mlp(partials, Wo1, bo1, Wo2, bo2):
    return pl.pallas_call(
        _out_body,
        out_shape=jax.ShapeDtypeStruct((_N, _D), jnp.float32),
    )(partials, Wo1, bo1, Wo2, bo2)


def kernel(x, pairlist, f_ij, f_ij_cutoff,
           W_in, Wf1, bf1, Wf2, bf2, Wo1, bo1, Wo2, bo2):
    h = _compute_h(x, W_in)
    wc = _compute_wc(f_ij.reshape(_E, _R), f_ij_cutoff,
                     Wf1, bf1.reshape(1, _F), Wf2, bf2.reshape(1, _F))
    idx_i = pairlist[0]
    idx_j = pairlist[1]
    zeros = jnp.zeros((_RPT, _D), jnp.float32)
    partials = _sc_aggregate(h, wc, idx_i, idx_j, zeros)
    out = _out_mlp(partials.reshape(_NC, _NP, _D),
                   Wo1, bo1.reshape(1, _D), Wo2, bo2.reshape(1, _D))
    return out
